# Initial kernel scaffold; baseline (speedup 1.0000x reference)
import jax
import jax.numpy as jnp
from jax.experimental import pallas as pl


def kernel(x, edge_index, edge_type, bn_gamma, bn_beta, W_root, W_rel, bias):
    raise NotImplementedError("write your pallas kernel here")


# speedup vs baseline: 10.6372x; 10.6372x over previous
import functools

import jax
import jax.numpy as jnp
from jax import lax
from jax.experimental import pallas as pl
from jax.experimental.pallas import tpu as pltpu
from jax.experimental.pallas import tpu_sc as plsc

_NGARB = 240
_K = 128
_NT = 16


def _bn_body(h_ref, g_ref, b_ref, o_ref):
    h = h_ref[...]
    m = jnp.mean(h, axis=0, keepdims=True)
    d = h - m
    v = jnp.mean(d * d, axis=0, keepdims=True)
    o_ref[...] = d * lax.rsqrt(v + 1e-5) * g_ref[...] + b_ref[...]


def _bn(h, gamma, beta):
    n, d = h.shape
    return pl.pallas_call(
        _bn_body,
        out_shape=jax.ShapeDtypeStruct((n, d), jnp.float32),
    )(h, gamma.reshape(1, d), beta.reshape(1, d))


def _combine_body(acc_ref, cnt_ref, hbn_ref, w0_ref, w1_ref, wr_ref, b_ref,
                  o_ref, *maybe_hprev):
    d = w0_ref.shape[0]
    r = 1.0 / jnp.maximum(cnt_ref[...], 1.0)
    m0 = acc_ref[:, :d] * r[:, 0:1]
    m1 = acc_ref[:, d:] * r[:, 1:2]
    z = (jnp.dot(m0, w0_ref[...], preferred_element_type=jnp.float32)
         + jnp.dot(m1, w1_ref[...], preferred_element_type=jnp.float32)
         + jnp.dot(hbn_ref[...], wr_ref[...], preferred_element_type=jnp.float32)
         + b_ref[...])
    z = jnp.maximum(z, 0.0)
    o_ref[...] = z


def _combine_res_body(acc_ref, cnt_ref, hbn_ref, hprev_ref, w0_ref, w1_ref,
                      wr_ref, b_ref, o_ref):
    d = w0_ref.shape[0]
    r = 1.0 / jnp.maximum(cnt_ref[...], 1.0)
    m0 = acc_ref[:, :d] * r[:, 0:1]
    m1 = acc_ref[:, d:] * r[:, 1:2]
    z = (jnp.dot(m0, w0_ref[...], preferred_element_type=jnp.float32)
         + jnp.dot(m1, w1_ref[...], preferred_element_type=jnp.float32)
         + jnp.dot(hbn_ref[...], wr_ref[...], preferred_element_type=jnp.float32)
         + b_ref[...])
    z = jnp.maximum(z, 0.0)
    o_ref[...] = hprev_ref[...] + z


def _combine(acc2, cnt2, hbn, hprev, w0, w1, wr, b):
    n, d = hbn.shape
    blk = 1000
    grid = (n // blk,)
    row_spec = lambda width: pl.BlockSpec((blk, width), lambda i: (i, 0))
    w_spec = pl.BlockSpec((d, d), lambda i: (0, 0))
    b_spec = pl.BlockSpec((1, d), lambda i: (0, 0))
    if hprev is None:
        in_specs = [row_spec(2 * d), row_spec(2), row_spec(d),
                    w_spec, w_spec, w_spec, b_spec]
        args = (acc2, cnt2, hbn, w0, w1, wr, b.reshape(1, d))
        body = _combine_body
    else:
        in_specs = [row_spec(2 * d), row_spec(2), row_spec(d), row_spec(d),
                    w_spec, w_spec, w_spec, b_spec]
        args = (acc2, cnt2, hbn, hprev, w0, w1, wr, b.reshape(1, d))
        body = _combine_res_body
    return pl.pallas_call(
        body,
        grid=grid,
        in_specs=in_specs,
        out_specs=row_spec(d),
        out_shape=jax.ShapeDtypeStruct((n, d), jnp.float32),
    )(*args)


@functools.lru_cache(maxsize=None)
def _make_sc_scatter(n, d, half, acc_rows, cpt, eptp, epad):
    nseg = 2 * half
    mesh = plsc.VectorSubcoreMesh(core_axis_name="c", subcore_axis_name="s")
    zrows = acc_rows // _NT // _K
    orows = (half // _NT) // 8 * 8
    otail = half - _NT * orows

    @functools.partial(
        pl.kernel,
        out_type=jax.ShapeDtypeStruct((nseg, d), jnp.float32),
        mesh=mesh,
        scratch_types=[
            pltpu.VMEM((_K,), jnp.int32),
            pltpu.VMEM((_K,), jnp.int32),
            pltpu.VMEM((_K, d), jnp.float32),
            pltpu.VMEM((_K, d), jnp.float32),
            pltpu.VMEM_SHARED((acc_rows, d), jnp.float32),
            pltpu.SemaphoreType.DMA,
        ],
    )
    def sc_scatter(h_hbm, src_hbm, loc_hbm, out_hbm,
                   srcbuf, segbuf, rowbuf, zbuf, acc, sem):
        c = lax.axis_index("c")
        s = lax.axis_index("s")
        zero16 = jnp.zeros((16,), jnp.float32)

        def zrow(i, carry):
            for j in range(d // 16):
                zbuf[i, pl.ds(j * 16, 16)] = zero16
            return carry
        lax.fori_loop(0, _K, zrow, 0)
        for i in range(zrows):
            pltpu.sync_copy(zbuf,
                            acc.at[pl.ds(s * (zrows * _K) + i * _K, _K)])
        plsc.subcore_barrier()

        base = s * eptp

        def chunk(g, carry):
            b = base + g * _K
            pltpu.sync_copy(src_hbm.at[pl.ds(b, _K)], srcbuf)
            pltpu.sync_copy(loc_hbm.at[pl.ds(c * epad + b, _K)], segbuf)
            pltpu.async_copy(h_hbm.at[srcbuf], rowbuf, sem).wait()
            pltpu.sync_copy(rowbuf, acc.at[segbuf], add=True)
            return carry
        lax.fori_loop(0, cpt, chunk, 0)
        plsc.subcore_barrier()

        pltpu.sync_copy(acc.at[pl.ds(s * orows, orows)],
                        out_hbm.at[pl.ds(c * half + s * orows, orows)])
        if otail:
            @pl.when(s == _NT - 1)
            def _():
                pltpu.sync_copy(
                    acc.at[pl.ds(_NT * orows, otail)],
                    out_hbm.at[pl.ds(c * half + _NT * orows, otail)])

    return sc_scatter


@functools.lru_cache(maxsize=None)
def _make_sc_counts(half, acc_rows, cpt, eptp, epad):
    mesh = plsc.VectorSubcoreMesh(core_axis_name="c", subcore_axis_name="s")
    zelems = acc_rows // _NT

    @functools.partial(
        pl.kernel,
        out_type=jax.ShapeDtypeStruct((2 * acc_rows,), jnp.float32),
        mesh=mesh,
        scratch_types=[
            pltpu.VMEM((_K,), jnp.int32),
            pltpu.VMEM((_K,), jnp.float32),
            pltpu.VMEM((zelems,), jnp.float32),
            pltpu.VMEM_SHARED((acc_rows,), jnp.float32),
        ],
    )
    def sc_counts(loc_hbm, out_hbm, segbuf, onesbuf, zbuf, acc):
        c = lax.axis_index("c")
        s = lax.axis_index("s")
        one16 = jnp.ones((16,), jnp.float32)
        zero16 = jnp.zeros((16,), jnp.float32)
        for j in range(_K // 16):
            onesbuf[pl.ds(j * 16, 16)] = one16

        def zfill(i, carry):
            zbuf[pl.ds(i * 16, 16)] = zero16
            return carry
        lax.fori_loop(0, zelems // 16, zfill, 0)
        pltpu.sync_copy(zbuf, acc.at[pl.ds(s * zelems, zelems)])
        plsc.subcore_barrier()

        base = s * eptp

        def chunk(g, carry):
            b = base + g * _K
            pltpu.sync_copy(loc_hbm.at[pl.ds(c * epad + b, _K)], segbuf)
            pltpu.sync_copy(onesbuf, acc.at[segbuf], add=True)
            return carry
        lax.fori_loop(0, cpt, chunk, 0)
        plsc.subcore_barrier()

        pltpu.sync_copy(acc.at[pl.ds(s * zelems, zelems)],
                        out_hbm.at[pl.ds(c * acc_rows + s * zelems, zelems)])

    return sc_counts


def kernel(x, edge_index, edge_type, bn_gamma, bn_beta, W_root, W_rel, bias):
    n, d = x.shape
    r = W_rel.shape[1]
    e = edge_index.shape[1]
    num_convs = W_root.shape[0]
    nseg = n * r
    half = nseg // 2
    acc_rows = half + _NGARB
    ept = e // _NT
    cpt = -(-ept // _K)
    eptp = cpt * _K
    epad = _NT * eptp

    src = edge_index[0].astype(jnp.int32)
    dst = edge_index[1].astype(jnp.int32)
    et = edge_type.astype(jnp.int32)
    seg = dst * r + et

    garb = half + (jnp.arange(e, dtype=jnp.int32) % _NGARB)
    gtail = half + (jnp.arange(e, epad, dtype=jnp.int32) % _NGARB)
    locs = []
    for c in range(2):
        sc = seg - c * half
        ok = (sc >= 0) & (sc < half)
        locs.append(jnp.concatenate([jnp.where(ok, sc, garb), gtail]))
    loc = jnp.concatenate(locs)
    srcp = jnp.concatenate([src, jnp.zeros(epad - e, jnp.int32)])

    cnt_raw = _make_sc_counts(half, acc_rows, cpt, eptp, epad)(loc)
    cnt2 = jnp.concatenate(
        [cnt_raw[:half], cnt_raw[acc_rows:acc_rows + half]]).reshape(n, r)

    scat = _make_sc_scatter(n, d, half, acc_rows, cpt, eptp, epad)

    h = x
    for i in range(num_convs):
        hbn = _bn(h, bn_gamma[i], bn_beta[i])
        acc = scat(hbn, srcp, loc)
        acc2 = acc.reshape(n, r * d)
        hprev = None if i == 0 else h
        h = _combine(acc2, cnt2, hbn, hprev,
                     W_rel[i, 0], W_rel[i, 1], W_root[i], bias[i])
    return h


# speedup vs baseline: 24.7342x; 2.3253x over previous
import functools

import jax
import jax.numpy as jnp
from jax import lax
from jax.experimental import pallas as pl
from jax.experimental.pallas import tpu as pltpu
from jax.experimental.pallas import tpu_sc as plsc

_K = 128
_NT = 16
_NBUF = 4
_CGARB = 480


def _bn_body(h_ref, g_ref, b_ref, o_ref):
    h = h_ref[...]
    m = jnp.mean(h, axis=0, keepdims=True)
    d = h - m
    v = jnp.mean(d * d, axis=0, keepdims=True)
    o_ref[...] = d * lax.rsqrt(v + 1e-5) * g_ref[...] + b_ref[...]


def _bn(h, gamma, beta):
    n, d = h.shape
    return pl.pallas_call(
        _bn_body,
        out_shape=jax.ShapeDtypeStruct((n, d), jnp.float32),
    )(h, gamma.reshape(1, d), beta.reshape(1, d))


def _combine_body(acc_ref, cnt_ref, hbn_ref, w0_ref, w1_ref, wr_ref, b_ref,
                  o_ref):
    d = w0_ref.shape[0]
    r = 1.0 / jnp.maximum(cnt_ref[...], 1.0)
    m0 = acc_ref[:, :d] * r[:, 0:1]
    m1 = acc_ref[:, d:] * r[:, 1:2]
    z = (jnp.dot(m0, w0_ref[...], preferred_element_type=jnp.float32)
         + jnp.dot(m1, w1_ref[...], preferred_element_type=jnp.float32)
         + jnp.dot(hbn_ref[...], wr_ref[...], preferred_element_type=jnp.float32)
         + b_ref[...])
    o_ref[...] = jnp.maximum(z, 0.0)


def _combine_res_body(acc_ref, cnt_ref, hbn_ref, hprev_ref, w0_ref, w1_ref,
                      wr_ref, b_ref, o_ref):
    d = w0_ref.shape[0]
    r = 1.0 / jnp.maximum(cnt_ref[...], 1.0)
    m0 = acc_ref[:, :d] * r[:, 0:1]
    m1 = acc_ref[:, d:] * r[:, 1:2]
    z = (jnp.dot(m0, w0_ref[...], preferred_element_type=jnp.float32)
         + jnp.dot(m1, w1_ref[...], preferred_element_type=jnp.float32)
         + jnp.dot(hbn_ref[...], wr_ref[...], preferred_element_type=jnp.float32)
         + b_ref[...])
    o_ref[...] = hprev_ref[...] + jnp.maximum(z, 0.0)


def _combine(acc2, cnt2, hbn, hprev, w0, w1, wr, b):
    n, d = hbn.shape
    blk = 1000
    grid = (n // blk,)
    row_spec = lambda width: pl.BlockSpec((blk, width), lambda i: (i, 0))
    w_spec = pl.BlockSpec((d, d), lambda i: (0, 0))
    b_spec = pl.BlockSpec((1, d), lambda i: (0, 0))
    if hprev is None:
        in_specs = [row_spec(2 * d), row_spec(2), row_spec(d),
                    w_spec, w_spec, w_spec, b_spec]
        args = (acc2, cnt2, hbn, w0, w1, wr, b.reshape(1, d))
        body = _combine_body
    else:
        in_specs = [row_spec(2 * d), row_spec(2), row_spec(d), row_spec(d),
                    w_spec, w_spec, w_spec, b_spec]
        args = (acc2, cnt2, hbn, hprev, w0, w1, wr, b.reshape(1, d))
        body = _combine_res_body
    return pl.pallas_call(
        body,
        grid=grid,
        in_specs=in_specs,
        out_specs=row_spec(d),
        out_shape=jax.ShapeDtypeStruct((n, d), jnp.float32),
    )(*args)


@functools.lru_cache(maxsize=None)
def _make_sc_scatter(n, d, half, acc_rows, cpt):
    mesh = plsc.VectorSubcoreMesh(core_axis_name="c", subcore_axis_name="s")
    zrows = acc_rows // _NT
    orows = (half // _NT) // 8 * 8
    otail = half - _NT * orows

    @functools.partial(
        pl.kernel,
        out_type=jax.ShapeDtypeStruct((2 * half, d), jnp.float32),
        mesh=mesh,
        scratch_types=[
            [pltpu.VMEM((2, _K), jnp.int32) for _ in range(4)],
            [pltpu.VMEM((_K, d), jnp.float32) for _ in range(2)],
            pltpu.VMEM_SHARED((acc_rows, d), jnp.float32),
            [pltpu.SemaphoreType.DMA for _ in range(4)],
            [pltpu.SemaphoreType.DMA for _ in range(2)],
        ],
    )
    def sc_scatter(h_hbm, comb_hbm, out_hbm, ibufs, rbufs, acc, isems, gsems):
        c = lax.axis_index("c")
        s = lax.axis_index("s")
        tbase = (c * _NT + s) * cpt

        def idx_issue(g, b):
            pltpu.async_copy(comb_hbm.at[pl.ds((tbase + g) * 2, 2)],
                             ibufs[b], isems[b])

        def idx_wait(b):
            pltpu.make_async_copy(comb_hbm.at[pl.ds(0, 2)],
                                  ibufs[b], isems[b]).wait()

        def gat_issue(b, b2):
            pltpu.async_copy(h_hbm.at[ibufs[b].at[0]], rbufs[b2], gsems[b2])

        def gat_wait(b, b2):
            pltpu.make_async_copy(h_hbm.at[ibufs[b].at[0]],
                                  rbufs[b2], gsems[b2]).wait()

        for g in range(4):
            idx_issue(g, g)

        zero16 = jnp.zeros((16,), jnp.float32)
        zbuf = rbufs[0]

        def zrow(i, carry):
            for j in range(d // 16):
                zbuf[i, pl.ds(j * 16, 16)] = zero16
            return carry
        lax.fori_loop(0, _K, zrow, 0)
        for i in range(zrows // _K):
            pltpu.sync_copy(zbuf, acc.at[pl.ds(s * zrows + i * _K, _K)])
        plsc.subcore_barrier()

        for g in range(2):
            idx_wait(g)
            gat_issue(g, g)

        def pipe(i, carry):
            for b in range(4):
                g = i * 4 + b
                b2 = b % 2
                gat_wait(b, b2)
                pltpu.sync_copy(rbufs[b2], acc.at[ibufs[b].at[1]], add=True)

                @pl.when(g + 4 < cpt)
                def _():
                    idx_issue(g + 4, b)

                @pl.when(g + 2 < cpt)
                def _():
                    bb = (b + 2) % 4
                    idx_wait(bb)
                    gat_issue(bb, b2)
            return carry
        lax.fori_loop(0, cpt // 4, pipe, 0)
        plsc.subcore_barrier()

        pltpu.sync_copy(acc.at[pl.ds(s * orows, orows)],
                        out_hbm.at[pl.ds(c * half + s * orows, orows)])
        if otail:
            @pl.when(s == _NT - 1)
            def _():
                pltpu.sync_copy(
                    acc.at[pl.ds(_NT * orows, otail)],
                    out_hbm.at[pl.ds(c * half + _NT * orows, otail)])

    return sc_scatter


@functools.lru_cache(maxsize=None)
def _make_sc_counts(acc_rows, cpt, eptp):
    mesh = plsc.VectorSubcoreMesh(core_axis_name="c", subcore_axis_name="s")
    zelems = acc_rows // _NT

    @functools.partial(
        pl.kernel,
        out_type=jax.ShapeDtypeStruct((2 * acc_rows,), jnp.float32),
        mesh=mesh,
        scratch_types=[
            pltpu.VMEM((_K,), jnp.int32),
            pltpu.VMEM((_K,), jnp.float32),
            pltpu.VMEM((zelems,), jnp.float32),
            pltpu.VMEM_SHARED((acc_rows,), jnp.float32),
        ],
    )
    def sc_counts(seg_hbm, out_hbm, segbuf, onesbuf, zbuf, acc):
        c = lax.axis_index("c")
        s = lax.axis_index("s")
        one16 = jnp.ones((16,), jnp.float32)
        zero16 = jnp.zeros((16,), jnp.float32)
        for j in range(_K // 16):
            onesbuf[pl.ds(j * 16, 16)] = one16

        def zfill(i, carry):
            zbuf[pl.ds(i * 16, 16)] = zero16
            return carry
        lax.fori_loop(0, zelems // 16, zfill, 0)
        pltpu.sync_copy(zbuf, acc.at[pl.ds(s * zelems, zelems)])
        plsc.subcore_barrier()

        base = s * eptp

        def chunk(g, carry):
            pltpu.sync_copy(seg_hbm.at[pl.ds(base + g * _K, _K)], segbuf)
            pltpu.sync_copy(onesbuf, acc.at[segbuf], add=True)
            return carry
        lax.fori_loop(0, cpt, chunk, 0)
        plsc.subcore_barrier()

        pltpu.sync_copy(acc.at[pl.ds(s * zelems, zelems)],
                        out_hbm.at[pl.ds(c * acc_rows + s * zelems, zelems)])

    return sc_counts


def kernel(x, edge_index, edge_type, bn_gamma, bn_beta, W_root, W_rel, bias):
    n, d = x.shape
    r = W_rel.shape[1]
    e = edge_index.shape[1]
    num_convs = W_root.shape[0]
    nseg = n * r
    half = nseg // 2
    acc_rows = 10240
    ngarb = acc_rows - half
    cnt_rows = nseg + _CGARB
    ept = e // _NT
    cpt = -(-(-(-ept // _K)) // _NBUF) * _NBUF
    eptp = cpt * _K
    epad = _NT * eptp

    src = edge_index[0].astype(jnp.int32)
    dst = edge_index[1].astype(jnp.int32)
    et = edge_type.astype(jnp.int32)
    seg = dst * r + et

    pad_src = jnp.arange(e, epad, dtype=jnp.int32) % n
    srcfull = jnp.concatenate([src, pad_src])
    src3 = jnp.broadcast_to(srcfull.reshape(_NT * cpt, _K),
                            (2, _NT * cpt, _K))

    gtail = nseg + (jnp.arange(e, epad, dtype=jnp.int32) % _CGARB)
    segfull = jnp.concatenate([seg, gtail])

    spread = jnp.arange(e, dtype=jnp.int32) % ngarb
    tail_spread = half + (jnp.arange(e, epad, dtype=jnp.int32) % ngarb)
    locs = []
    for c in range(2):
        sc = seg - c * half
        ok = (sc >= 0) & (sc < half)
        locs.append(jnp.concatenate([jnp.where(ok, sc, half + spread),
                                     tail_spread]))
    loc3 = jnp.concatenate(locs).reshape(2, _NT * cpt, _K)
    comb = jnp.stack([src3, loc3], axis=2).reshape(2 * _NT * cpt * 2, _K)

    cnt_raw = _make_sc_counts(cnt_rows, cpt, eptp)(segfull)
    cnt2 = cnt_raw[:nseg].reshape(n, r)

    scat = _make_sc_scatter(n, d, half, acc_rows, cpt)

    h = x
    for i in range(num_convs):
        hbn = _bn(h, bn_gamma[i], bn_beta[i])
        acc = scat(hbn, comb)
        acc2 = acc.reshape(n, r * d)
        hprev = None if i == 0 else h
        h = _combine(acc2, cnt2, hbn, hprev,
                     W_rel[i, 0], W_rel[i, 1], W_root[i], bias[i])
    return h


# speedup vs baseline: 25.7565x; 1.0413x over previous
import functools

import jax
import jax.numpy as jnp
from jax import lax
from jax.experimental import pallas as pl
from jax.experimental.pallas import tpu as pltpu
from jax.experimental.pallas import tpu_sc as plsc

_K = 128
_NT = 16
_NBUF = 4
_CGARB = 480


def _bn_body(h_ref, g_ref, b_ref, o_ref):
    h = h_ref[...]
    m = jnp.mean(h, axis=0, keepdims=True)
    d = h - m
    v = jnp.mean(d * d, axis=0, keepdims=True)
    o_ref[...] = d * lax.rsqrt(v + 1e-5) * g_ref[...] + b_ref[...]


def _bn(h, gamma, beta):
    n, d = h.shape
    return pl.pallas_call(
        _bn_body,
        out_shape=jax.ShapeDtypeStruct((n, d), jnp.float32),
    )(h, gamma.reshape(1, d), beta.reshape(1, d))


def _combine_body(acc_ref, cnt_ref, hbn_ref, w0_ref, w1_ref, wr_ref, b_ref,
                  o_ref):
    d = w0_ref.shape[0]
    r = 1.0 / jnp.maximum(cnt_ref[...], 1.0)
    m0 = acc_ref[:, :d] * r[:, 0:1]
    m1 = acc_ref[:, d:] * r[:, 1:2]
    z = (jnp.dot(m0, w0_ref[...], preferred_element_type=jnp.float32)
         + jnp.dot(m1, w1_ref[...], preferred_element_type=jnp.float32)
         + jnp.dot(hbn_ref[...], wr_ref[...], preferred_element_type=jnp.float32)
         + b_ref[...])
    o_ref[...] = jnp.maximum(z, 0.0)


def _combine_res_body(acc_ref, cnt_ref, hbn_ref, hprev_ref, w0_ref, w1_ref,
                      wr_ref, b_ref, o_ref):
    d = w0_ref.shape[0]
    r = 1.0 / jnp.maximum(cnt_ref[...], 1.0)
    m0 = acc_ref[:, :d] * r[:, 0:1]
    m1 = acc_ref[:, d:] * r[:, 1:2]
    z = (jnp.dot(m0, w0_ref[...], preferred_element_type=jnp.float32)
         + jnp.dot(m1, w1_ref[...], preferred_element_type=jnp.float32)
         + jnp.dot(hbn_ref[...], wr_ref[...], preferred_element_type=jnp.float32)
         + b_ref[...])
    o_ref[...] = hprev_ref[...] + jnp.maximum(z, 0.0)


def _combine_bn_body(acc_ref, cnt_ref, hbn_ref, w0_ref, w1_ref, wr_ref, b_ref,
                     g2_ref, b2_ref, o_ref, obn_ref):
    d = w0_ref.shape[0]
    r = 1.0 / jnp.maximum(cnt_ref[...], 1.0)
    m0 = acc_ref[:, :d] * r[:, 0:1]
    m1 = acc_ref[:, d:] * r[:, 1:2]
    z = (jnp.dot(m0, w0_ref[...], preferred_element_type=jnp.float32)
         + jnp.dot(m1, w1_ref[...], preferred_element_type=jnp.float32)
         + jnp.dot(hbn_ref[...], wr_ref[...], preferred_element_type=jnp.float32)
         + b_ref[...])
    h = jnp.maximum(z, 0.0)
    o_ref[...] = h
    m = jnp.mean(h, axis=0, keepdims=True)
    dlt = h - m
    v = jnp.mean(dlt * dlt, axis=0, keepdims=True)
    obn_ref[...] = dlt * lax.rsqrt(v + 1e-5) * g2_ref[...] + b2_ref[...]


def _combine_bn_res_body(acc_ref, cnt_ref, hbn_ref, hprev_ref, w0_ref, w1_ref,
                         wr_ref, b_ref, g2_ref, b2_ref, o_ref, obn_ref):
    d = w0_ref.shape[0]
    r = 1.0 / jnp.maximum(cnt_ref[...], 1.0)
    m0 = acc_ref[:, :d] * r[:, 0:1]
    m1 = acc_ref[:, d:] * r[:, 1:2]
    z = (jnp.dot(m0, w0_ref[...], preferred_element_type=jnp.float32)
         + jnp.dot(m1, w1_ref[...], preferred_element_type=jnp.float32)
         + jnp.dot(hbn_ref[...], wr_ref[...], preferred_element_type=jnp.float32)
         + b_ref[...])
    h = hprev_ref[...] + jnp.maximum(z, 0.0)
    o_ref[...] = h
    m = jnp.mean(h, axis=0, keepdims=True)
    dlt = h - m
    v = jnp.mean(dlt * dlt, axis=0, keepdims=True)
    obn_ref[...] = dlt * lax.rsqrt(v + 1e-5) * g2_ref[...] + b2_ref[...]


def _combine_bn(acc2, cnt2, hbn, hprev, w0, w1, wr, b, g2, b2):
    n, d = hbn.shape
    out_shape = [jax.ShapeDtypeStruct((n, d), jnp.float32),
                 jax.ShapeDtypeStruct((n, d), jnp.float32)]
    if hprev is None:
        args = (acc2, cnt2, hbn, w0, w1, wr, b.reshape(1, d),
                g2.reshape(1, d), b2.reshape(1, d))
        body = _combine_bn_body
    else:
        args = (acc2, cnt2, hbn, hprev, w0, w1, wr, b.reshape(1, d),
                g2.reshape(1, d), b2.reshape(1, d))
        body = _combine_bn_res_body
    return pl.pallas_call(body, out_shape=out_shape)(*args)


def _combine(acc2, cnt2, hbn, hprev, w0, w1, wr, b):
    n, d = hbn.shape
    blk = 1000
    grid = (n // blk,)
    row_spec = lambda width: pl.BlockSpec((blk, width), lambda i: (i, 0))
    w_spec = pl.BlockSpec((d, d), lambda i: (0, 0))
    b_spec = pl.BlockSpec((1, d), lambda i: (0, 0))
    if hprev is None:
        in_specs = [row_spec(2 * d), row_spec(2), row_spec(d),
                    w_spec, w_spec, w_spec, b_spec]
        args = (acc2, cnt2, hbn, w0, w1, wr, b.reshape(1, d))
        body = _combine_body
    else:
        in_specs = [row_spec(2 * d), row_spec(2), row_spec(d), row_spec(d),
                    w_spec, w_spec, w_spec, b_spec]
        args = (acc2, cnt2, hbn, hprev, w0, w1, wr, b.reshape(1, d))
        body = _combine_res_body
    return pl.pallas_call(
        body,
        grid=grid,
        in_specs=in_specs,
        out_specs=row_spec(d),
        out_shape=jax.ShapeDtypeStruct((n, d), jnp.float32),
    )(*args)


@functools.lru_cache(maxsize=None)
def _make_sc_scatter(n, d, half, acc_rows, cpt):
    mesh = plsc.VectorSubcoreMesh(core_axis_name="c", subcore_axis_name="s")
    zrows = acc_rows // _NT
    orows = (half // _NT) // 8 * 8
    otail = half - _NT * orows

    @functools.partial(
        pl.kernel,
        out_type=jax.ShapeDtypeStruct((2 * half, d), jnp.float32),
        mesh=mesh,
        scratch_types=[
            [pltpu.VMEM((2, _K), jnp.int32) for _ in range(4)],
            [pltpu.VMEM((_K, d), jnp.float32) for _ in range(2)],
            pltpu.VMEM_SHARED((acc_rows, d), jnp.float32),
            [pltpu.SemaphoreType.DMA for _ in range(4)],
            [pltpu.SemaphoreType.DMA for _ in range(2)],
        ],
    )
    def sc_scatter(h_hbm, comb_hbm, out_hbm, ibufs, rbufs, acc, isems, gsems):
        c = lax.axis_index("c")
        s = lax.axis_index("s")
        tbase = (c * _NT + s) * cpt

        def idx_issue(g, b):
            pltpu.async_copy(comb_hbm.at[pl.ds((tbase + g) * 2, 2)],
                             ibufs[b], isems[b])

        def idx_wait(b):
            pltpu.make_async_copy(comb_hbm.at[pl.ds(0, 2)],
                                  ibufs[b], isems[b]).wait()

        def gat_issue(b, b2):
            pltpu.async_copy(h_hbm.at[ibufs[b].at[0]], rbufs[b2], gsems[b2])

        def gat_wait(b, b2):
            pltpu.make_async_copy(h_hbm.at[ibufs[b].at[0]],
                                  rbufs[b2], gsems[b2]).wait()

        for g in range(4):
            idx_issue(g, g)

        zero16 = jnp.zeros((16,), jnp.float32)
        zbuf = rbufs[0]

        def zrow(i, carry):
            for j in range(d // 16):
                zbuf[i, pl.ds(j * 16, 16)] = zero16
            return carry
        lax.fori_loop(0, _K, zrow, 0)
        for i in range(zrows // _K):
            pltpu.sync_copy(zbuf, acc.at[pl.ds(s * zrows + i * _K, _K)])
        plsc.subcore_barrier()

        for g in range(2):
            idx_wait(g)
            gat_issue(g, g)

        def pipe(i, carry):
            for b in range(4):
                g = i * 4 + b
                b2 = b % 2
                gat_wait(b, b2)
                pltpu.sync_copy(rbufs[b2], acc.at[ibufs[b].at[1]], add=True)

                @pl.when(g + 4 < cpt)
                def _():
                    idx_issue(g + 4, b)

                @pl.when(g + 2 < cpt)
                def _():
                    bb = (b + 2) % 4
                    idx_wait(bb)
                    gat_issue(bb, b2)
            return carry
        lax.fori_loop(0, cpt // 4, pipe, 0)
        plsc.subcore_barrier()

        pltpu.sync_copy(acc.at[pl.ds(s * orows, orows)],
                        out_hbm.at[pl.ds(c * half + s * orows, orows)])
        if otail:
            @pl.when(s == _NT - 1)
            def _():
                pltpu.sync_copy(
                    acc.at[pl.ds(_NT * orows, otail)],
                    out_hbm.at[pl.ds(c * half + _NT * orows, otail)])

    return sc_scatter


@functools.lru_cache(maxsize=None)
def _make_sc_counts(acc_rows, cpt):
    mesh = plsc.VectorSubcoreMesh(core_axis_name="c", subcore_axis_name="s")
    zelems = acc_rows // _NT

    @functools.partial(
        pl.kernel,
        out_type=jax.ShapeDtypeStruct((2 * acc_rows,), jnp.float32),
        mesh=mesh,
        scratch_types=[
            [pltpu.VMEM((1, _K), jnp.int32) for _ in range(4)],
            pltpu.VMEM((_K,), jnp.float32),
            pltpu.VMEM((zelems,), jnp.float32),
            pltpu.VMEM_SHARED((acc_rows,), jnp.float32),
            [pltpu.SemaphoreType.DMA for _ in range(4)],
            [pltpu.SemaphoreType.DMA for _ in range(4)],
        ],
    )
    def sc_counts(seg_hbm, out_hbm, segbufs, onesbuf, zbuf, acc, isems, ssems):
        c = lax.axis_index("c")
        s = lax.axis_index("s")
        tbase = s * cpt

        def idx_issue(g, b):
            pltpu.async_copy(seg_hbm.at[pl.ds(tbase + g, 1)],
                             segbufs[b], isems[b])

        def idx_wait(b):
            pltpu.make_async_copy(seg_hbm.at[pl.ds(0, 1)],
                                  segbufs[b], isems[b]).wait()

        def sc_wait(b):
            pltpu.make_async_copy(onesbuf, acc.at[segbufs[b].at[0]],
                                  ssems[b]).wait()

        for g in range(4):
            idx_issue(g, g)

        one16 = jnp.ones((16,), jnp.float32)
        zero16 = jnp.zeros((16,), jnp.float32)
        for j in range(_K // 16):
            onesbuf[pl.ds(j * 16, 16)] = one16

        def zfill(i, carry):
            zbuf[pl.ds(i * 16, 16)] = zero16
            return carry
        lax.fori_loop(0, zelems // 16, zfill, 0)
        pltpu.sync_copy(zbuf, acc.at[pl.ds(s * zelems, zelems)])
        plsc.subcore_barrier()

        def chunk(i, carry):
            for b in range(4):
                g = i * 4 + b
                idx_wait(b)
                pltpu.async_copy(onesbuf, acc.at[segbufs[b].at[0]],
                                 ssems[b], add=True)
                nb = (b + 1) % 4
                @pl.when((g + 1 >= 4) & (g + 1 < cpt))
                def _():
                    sc_wait(nb)
                    idx_issue(g + 1, nb)
            return carry
        lax.fori_loop(0, cpt // 4, chunk, 0)
        for bb in range(4):
            sc_wait(bb)
        plsc.subcore_barrier()

        pltpu.sync_copy(acc.at[pl.ds(s * zelems, zelems)],
                        out_hbm.at[pl.ds(c * acc_rows + s * zelems, zelems)])

    return sc_counts


def kernel(x, edge_index, edge_type, bn_gamma, bn_beta, W_root, W_rel, bias):
    n, d = x.shape
    r = W_rel.shape[1]
    e = edge_index.shape[1]
    num_convs = W_root.shape[0]
    nseg = n * r
    half = nseg // 2
    acc_rows = 10240
    ngarb = acc_rows - half
    cnt_rows = nseg + _CGARB
    ept = e // _NT
    cpt = -(-(-(-ept // _K)) // _NBUF) * _NBUF
    eptp = cpt * _K
    epad = _NT * eptp

    src = edge_index[0].astype(jnp.int32)
    dst = edge_index[1].astype(jnp.int32)
    et = edge_type.astype(jnp.int32)
    seg = dst * r + et

    pad_src = jnp.arange(e, epad, dtype=jnp.int32) % n
    srcfull = jnp.concatenate([src, pad_src])
    src3 = jnp.broadcast_to(srcfull.reshape(_NT * cpt, _K),
                            (2, _NT * cpt, _K))

    gtail = nseg + (jnp.arange(e, epad, dtype=jnp.int32) % _CGARB)
    segfull = jnp.concatenate([seg, gtail])
    seg2d = segfull.reshape(_NT * cpt, _K)

    spread = jnp.arange(e, dtype=jnp.int32) % ngarb
    tail_spread = half + (jnp.arange(e, epad, dtype=jnp.int32) % ngarb)
    locs = []
    for c in range(2):
        sc = seg - c * half
        ok = (sc >= 0) & (sc < half)
        locs.append(jnp.concatenate([jnp.where(ok, sc, half + spread),
                                     tail_spread]))
    loc3 = jnp.concatenate(locs).reshape(2, _NT * cpt, _K)
    comb = jnp.stack([src3, loc3], axis=2).reshape(2 * _NT * cpt * 2, _K)

    cnt_raw = _make_sc_counts(cnt_rows, cpt)(seg2d)
    cnt2 = cnt_raw[:nseg].reshape(n, r)

    scat = _make_sc_scatter(n, d, half, acc_rows, cpt)

    hprev = None
    hbn = _bn(x, bn_gamma[0], bn_beta[0])
    for i in range(num_convs):
        acc = scat(hbn, comb)
        acc2 = acc.reshape(n, r * d)
        if i + 1 < num_convs:
            h, hbn = _combine_bn(acc2, cnt2, hbn, hprev,
                                 W_rel[i, 0], W_rel[i, 1], W_root[i], bias[i],
                                 bn_gamma[i + 1], bn_beta[i + 1])
        else:
            h = _combine(acc2, cnt2, hbn, hprev,
                         W_rel[i, 0], W_rel[i, 1], W_root[i], bias[i])
        hprev = h
    return h
